# baseline (speedup 1.0000x reference)
import functools

import jax
import jax.numpy as jnp
from jax import lax
from jax.experimental import pallas as pl
from jax.experimental.pallas import tpu as pltpu
from jax.experimental.pallas import tpu_sc as plsc

N = 100000
R = 4
E = 400000
F = 128
NSL = 8
NSC = 2
NT = 16
EP = E // NT
BATCH = 128
CHK = 5
NCHK = 40
NBAT = NCHK * CHK
EPP = NBAT * BATCH
ACC_CHUNK = 6264
ACC_ROWS = ACC_CHUNK * NT
OUT_CHUNK = 6248
OUT_LAST = N - 15 * OUT_CHUNK
BN = 1024
NGRID = (N + BN - 1) // BN

_MESH = plsc.VectorSubcoreMesh(
    core_axis_name="c", subcore_axis_name="s", num_cores=NSC, num_subcores=NT
)


def _fill(ref, nrows, value):
    def body(i, _):
        ref[i, :] = jnp.full((16,), value, jnp.float32)
        return _
    lax.fori_loop(0, nrows, body, None)


def _zero_acc_chunk(zeros_v, acc, base):
    def zbody(i, carry):
        pltpu.sync_copy(zeros_v, acc.at[pl.ds(base + i * 256, 256)])
        return carry
    lax.fori_loop(0, 24, zbody, None)
    pltpu.sync_copy(zeros_v.at[pl.ds(0, ACC_CHUNK - 6144)],
                    acc.at[pl.ds(base + 6144, ACC_CHUNK - 6144)])


def _copy_out_chunk(t, src_ref, dst_ref):
    obase = t * OUT_CHUNK

    @pl.when(t < NT - 1)
    def _():
        pltpu.sync_copy(src_ref.at[pl.ds(obase, OUT_CHUNK)],
                        dst_ref.at[pl.ds(obase, OUT_CHUNK)])

    @pl.when(t == NT - 1)
    def _():
        pltpu.sync_copy(src_ref.at[pl.ds((NT - 1) * OUT_CHUNK, OUT_LAST)],
                        dst_ref.at[pl.ds((NT - 1) * OUT_CHUNK, OUT_LAST)])


def _copy_out_chunk_packed(t, acc, dst_ref):
    obase = t * OUT_CHUNK

    @pl.when(t < NT - 1)
    def _():
        pltpu.sync_copy(
            acc.at[pl.ds(obase, OUT_CHUNK)].reshape(OUT_CHUNK // 8, F),
            dst_ref.at[pl.ds(obase // 8, OUT_CHUNK // 8)])

    @pl.when(t == NT - 1)
    def _():
        pltpu.sync_copy(
            acc.at[pl.ds((NT - 1) * OUT_CHUNK, OUT_LAST)].reshape(
                OUT_LAST // 8, F),
            dst_ref.at[pl.ds((NT - 1) * OUT_CHUNK // 8, OUT_LAST // 8)])


@functools.partial(
    pl.kernel,
    out_type=jax.ShapeDtypeStruct((R, N, 16), jnp.float32),
    mesh=_MESH,
    compiler_params=pltpu.CompilerParams(use_tc_tiling_on_sc=False),
    scratch_types=[
        pltpu.VMEM((CHK, BATCH), jnp.int32),
        pltpu.VMEM((BATCH, 16), jnp.float32),
        pltpu.VMEM((256, 16), jnp.float32),
        pltpu.VMEM_SHARED((ACC_ROWS, 16), jnp.float32),
    ],
)
def _deg(epk_hbm, deg_hbm, dst_v, ones_v, zeros_v, acc):
    c = lax.axis_index("c")
    t = lax.axis_index("s")
    _fill(zeros_v, 256, 0.0)
    _fill(ones_v, BATCH, 1.0)
    for rl in range(R // NSC):
        r = c * (R // NSC) + rl
        _zero_acc_chunk(zeros_v, acc, t * ACC_CHUNK)
        plsc.subcore_barrier()

        def body(nc, carry):
            pltpu.sync_copy(epk_hbm.at[r, t, nc, 1], dst_v)
            for b in range(CHK):
                pltpu.sync_copy(ones_v, acc.at[dst_v.at[b]], add=True)
            return carry
        lax.fori_loop(0, NCHK, body, None)
        plsc.subcore_barrier()
        _copy_out_chunk(t, acc, deg_hbm.at[r])
        plsc.subcore_barrier()


def _make_agg(r):
    @functools.partial(
        pl.kernel,
        out_type=jax.ShapeDtypeStruct((NSL, N, 16), jnp.float32),
        mesh=_MESH,
        compiler_params=pltpu.CompilerParams(use_tc_tiling_on_sc=False),
        scratch_types=[
            pltpu.VMEM((2, 2, CHK, BATCH), jnp.int32),
            pltpu.VMEM((2, CHK * BATCH, 16), jnp.float32),
            pltpu.VMEM((256, 16), jnp.float32),
            pltpu.VMEM_SHARED((ACC_ROWS, 16), jnp.float32),
            pltpu.SemaphoreType.DMA,
            pltpu.SemaphoreType.DMA,
            pltpu.SemaphoreType.DMA,
            pltpu.SemaphoreType.DMA,
        ],
        name=f"agg_etype{r}",
    )
    def _agg_r(x_hbm, epk_hbm, y_hbm,
               idx_v, rows_v, zeros_v, acc,
               gsem0, gsem1, ssem0, ssem1):
        c = lax.axis_index("c")
        t = lax.axis_index("s")
        _fill(zeros_v, 256, 0.0)

        def combo(q, carry):
            s = c * (NSL // NSC) + q
            xs = x_hbm.at[s]
            _zero_acc_chunk(zeros_v, acc, t * ACC_CHUNK)
            plsc.subcore_barrier()

            def fire_gathers(nc, p, gsem):
                pltpu.sync_copy(epk_hbm.at[r, t, nc], idx_v.at[p])
                for b in range(CHK):
                    pltpu.async_copy(
                        xs.at[idx_v.at[p, 0, b]],
                        rows_v.at[p, pl.ds(b * BATCH, BATCH)], gsem)

            def drain_scatter(nc, p, gsem, ssem):
                for b in range(CHK):
                    pltpu.make_async_copy(
                        xs.at[idx_v.at[p, 0, b]],
                        rows_v.at[p, pl.ds(b * BATCH, BATCH)], gsem).wait()
                for b in range(CHK):
                    pltpu.async_copy(rows_v.at[p, pl.ds(b * BATCH, BATCH)],
                                     acc.at[idx_v.at[p, 1, b]], ssem,
                                     add=True)
                for b in range(CHK):
                    pltpu.make_async_copy(
                        rows_v.at[p, pl.ds(b * BATCH, BATCH)],
                        acc.at[idx_v.at[p, 1, b]], ssem).wait()

            fire_gathers(0, 0, gsem0)

            def pair(i, carry2):
                base = 2 * i
                fire_gathers(base + 1, 1, gsem1)
                drain_scatter(base, 0, gsem0, ssem0)

                @pl.when(base + 2 < NCHK)
                def _():
                    fire_gathers(base + 2, 0, gsem0)
                drain_scatter(base + 1, 1, gsem1, ssem1)
                return carry2
            lax.fori_loop(0, NCHK // 2, pair, None)
            plsc.subcore_barrier()
            _copy_out_chunk(t, acc, y_hbm.at[s])
            plsc.subcore_barrier()
            return carry

        lax.fori_loop(0, NSL // NSC, combo, None)
    return _agg_r


_AGGS = [_make_agg(r) for r in range(R)]


@functools.partial(
    pl.kernel,
    out_type=jax.ShapeDtypeStruct((128, F), jnp.float32),
    mesh=_MESH,
    scratch_types=[
        pltpu.VMEM((BATCH,), jnp.int32),
        pltpu.VMEM((BATCH, F), jnp.float32),
        pltpu.SemaphoreType.DMA,
    ],
)
def _take_rows(x_hbm, idx_hbm, out_hbm, idx_v, rows_v, sem):
    c = lax.axis_index("c")
    t = lax.axis_index("s")

    @pl.when(jnp.logical_and(c == 0, t == 0))
    def _():
        pltpu.sync_copy(idx_hbm, idx_v)
        pltpu.async_copy(x_hbm.at[idx_v], rows_v, sem).wait()
        pltpu.sync_copy(rows_v, out_hbm)


def _part_body(y_ref, deg_ref, w_ref, b_ref, o_ref, *, r):
    d = deg_ref[r, :, 0]
    rd = 1.0 / jnp.maximum(d, 1.0)
    yr = jnp.concatenate([y_ref[s] for s in range(NSL)], axis=1)
    yr = yr * rd[:, None]
    p = jnp.dot(yr, w_ref[r], preferred_element_type=jnp.float32)
    o_ref[...] = p + jnp.where(d > 0, 1.0, 0.0)[:, None] * b_ref[r][None, :]


def _part_tc(Y, deg, W, b, r):
    return pl.pallas_call(
        functools.partial(_part_body, r=r),
        grid=(NGRID,),
        in_specs=[
            pl.BlockSpec((NSL, BN, 16), lambda i: (0, i, 0)),
            pl.BlockSpec((R, BN, 16), lambda i: (0, i, 0)),
            pl.BlockSpec((R, F, F), lambda i: (0, 0, 0)),
            pl.BlockSpec((R, F), lambda i: (0, 0)),
        ],
        out_specs=pl.BlockSpec((BN, F), lambda i: (i, 0)),
        out_shape=jax.ShapeDtypeStruct((N, F), jnp.float32),
    )(Y, deg, W, b)


def _combine_body(p0, p1, p2, p3, o_ref, *, act, sliced):
    acc = act(p0[...] + p1[...] + p2[...] + p3[...])
    if sliced:
        for s in range(NSL):
            o_ref[s] = acc[:, s * 16:(s + 1) * 16]
    else:
        o_ref[...] = acc


def _combine_tc(Ps, act, sliced):
    if sliced:
        out_spec = pl.BlockSpec((NSL, BN, 16), lambda i: (0, i, 0))
        out_shape = jax.ShapeDtypeStruct((NSL, N, 16), jnp.float32)
    else:
        out_spec = pl.BlockSpec((BN, F), lambda i: (i, 0))
        out_shape = jax.ShapeDtypeStruct((N, F), jnp.float32)
    p_spec = pl.BlockSpec((BN, F), lambda i: (i, 0))
    return pl.pallas_call(
        functools.partial(_combine_body, act=act, sliced=sliced),
        grid=(NGRID,),
        in_specs=[p_spec, p_spec, p_spec, p_spec],
        out_specs=out_spec,
        out_shape=out_shape,
    )(*Ps)


def _slice_body(x_ref, o_ref):
    for s in range(NSL):
        o_ref[s] = x_ref[:, s * 16:(s + 1) * 16]


def _slice_tc(x):
    return pl.pallas_call(
        _slice_body,
        grid=(NGRID,),
        in_specs=[pl.BlockSpec((BN, F), lambda i: (i, 0))],
        out_specs=pl.BlockSpec((NSL, BN, 16), lambda i: (0, i, 0)),
        out_shape=jax.ShapeDtypeStruct((NSL, N, 16), jnp.float32),
    )(x)


def _lrelu(x):
    return jnp.where(x > 0, x, 0.01 * x)


def _final_body(ent_ref, rows_ref, wu_ref, bu_ref, o_ref):
    le = jax.nn.sigmoid(jnp.sum(rows_ref[:64], axis=0, keepdims=True))
    de = jax.nn.sigmoid(jnp.sum(rows_ref[64:], axis=0, keepdims=True))
    u = jax.nn.sigmoid(
        jnp.dot(le, wu_ref[:F], preferred_element_type=jnp.float32)
        + jnp.dot(de, wu_ref[F:], preferred_element_type=jnp.float32)
        + bu_ref[...])
    o_ref[...] = jnp.dot(ent_ref[...], u.reshape(F, 1),
                         preferred_element_type=jnp.float32)


def _final_tc(entity, rows, Wu, bu):
    return pl.pallas_call(
        _final_body,
        grid=(NGRID,),
        in_specs=[
            pl.BlockSpec((BN, F), lambda i: (i, 0)),
            pl.BlockSpec((128, F), lambda i: (0, 0)),
            pl.BlockSpec((2 * F, F), lambda i: (0, 0)),
            pl.BlockSpec((1, F), lambda i: (0, 0)),
        ],
        out_specs=pl.BlockSpec((BN, 1), lambda i: (i, 0)),
        out_shape=jax.ShapeDtypeStruct((N, 1), jnp.float32),
    )(entity, rows, Wu, bu)


def kernel(embed, W1, b1, W2, b2, W3, b3, Wu, bu,
           edge_src, edge_dst, liked_indices, disliked_indices):
    es = edge_src.astype(jnp.int32).reshape(R, NT, EP)
    ed = edge_dst.astype(jnp.int32).reshape(R, NT, EP)
    src_p = jnp.pad(es, ((0, 0), (0, 0), (0, EPP - EP))
                    ).reshape(R, NT, NCHK, CHK, BATCH)
    dst_p = jnp.pad(ed, ((0, 0), (0, 0), (0, EPP - EP)), constant_values=N
                    ).reshape(R, NT, NCHK, CHK, BATCH)
    epk = jnp.stack((src_p, dst_p), axis=3)

    deg = _deg(epk)

    x8 = _slice_tc(embed)

    def layer(x, W, b, act, sliced):
        Ps = [_part_tc(_AGGS[r](x, epk), deg, W, b, r) for r in range(R)]
        return _combine_tc(Ps, act, sliced)

    x8 = layer(x8, W1, b1, _lrelu, True)
    x8 = layer(x8, W2, b2, _lrelu, True)
    entity = layer(x8, W3, b3, jax.nn.sigmoid, False)

    idx = jnp.concatenate([liked_indices, disliked_indices]).astype(jnp.int32)
    rows = _take_rows(entity, idx)
    preds = _final_tc(entity, rows, Wu, bu.reshape(1, F))
    return preds.reshape(N)


# speedup vs baseline: 1.0579x; 1.0579x over previous
import functools

import jax
import jax.numpy as jnp
from jax import lax
from jax.experimental import pallas as pl
from jax.experimental.pallas import tpu as pltpu
from jax.experimental.pallas import tpu_sc as plsc

N = 100000
R = 4
E = 400000
F = 128
NSL = 8
NSC = 2
NT = 16
EP = E // NT
BATCH = 128
CHK = 5
NCHK = 40
NBAT = NCHK * CHK
EPP = NBAT * BATCH
ACC_CHUNK = 6264
ACC_ROWS = ACC_CHUNK * NT
OUT_CHUNK = 6248
OUT_LAST = N - 15 * OUT_CHUNK
BN = 1024
NGRID = (N + BN - 1) // BN

_MESH = plsc.VectorSubcoreMesh(
    core_axis_name="c", subcore_axis_name="s", num_cores=NSC, num_subcores=NT
)


def _fill(ref, nrows, value):
    def body(i, _):
        ref[i, :] = jnp.full((16,), value, jnp.float32)
        return _
    lax.fori_loop(0, nrows, body, None)


def _zero_acc_chunk(zeros_v, acc, base):
    def zbody(i, carry):
        pltpu.sync_copy(zeros_v, acc.at[pl.ds(base + i * 256, 256)])
        return carry
    lax.fori_loop(0, 24, zbody, None)
    pltpu.sync_copy(zeros_v.at[pl.ds(0, ACC_CHUNK - 6144)],
                    acc.at[pl.ds(base + 6144, ACC_CHUNK - 6144)])


def _copy_out_chunk(t, src_ref, dst_ref):
    obase = t * OUT_CHUNK

    @pl.when(t < NT - 1)
    def _():
        pltpu.sync_copy(src_ref.at[pl.ds(obase, OUT_CHUNK)],
                        dst_ref.at[pl.ds(obase, OUT_CHUNK)])

    @pl.when(t == NT - 1)
    def _():
        pltpu.sync_copy(src_ref.at[pl.ds((NT - 1) * OUT_CHUNK, OUT_LAST)],
                        dst_ref.at[pl.ds((NT - 1) * OUT_CHUNK, OUT_LAST)])


def _copy_out_chunk_packed(t, acc, dst_ref):
    obase = t * OUT_CHUNK

    @pl.when(t < NT - 1)
    def _():
        pltpu.sync_copy(
            acc.at[pl.ds(obase, OUT_CHUNK)].reshape(OUT_CHUNK // 8, F),
            dst_ref.at[pl.ds(obase // 8, OUT_CHUNK // 8)])

    @pl.when(t == NT - 1)
    def _():
        pltpu.sync_copy(
            acc.at[pl.ds((NT - 1) * OUT_CHUNK, OUT_LAST)].reshape(
                OUT_LAST // 8, F),
            dst_ref.at[pl.ds((NT - 1) * OUT_CHUNK // 8, OUT_LAST // 8)])


@functools.partial(
    pl.kernel,
    out_type=jax.ShapeDtypeStruct((R, N, 16), jnp.float32),
    mesh=_MESH,
    compiler_params=pltpu.CompilerParams(use_tc_tiling_on_sc=False),
    scratch_types=[
        pltpu.VMEM((CHK, BATCH), jnp.int32),
        pltpu.VMEM((BATCH, 16), jnp.float32),
        pltpu.VMEM((256, 16), jnp.float32),
        pltpu.VMEM_SHARED((ACC_ROWS, 16), jnp.float32),
    ],
)
def _deg(epk_hbm, deg_hbm, dst_v, ones_v, zeros_v, acc):
    c = lax.axis_index("c")
    t = lax.axis_index("s")
    _fill(zeros_v, 256, 0.0)
    _fill(ones_v, BATCH, 1.0)
    for rl in range(R // NSC):
        r = c * (R // NSC) + rl
        _zero_acc_chunk(zeros_v, acc, t * ACC_CHUNK)
        plsc.subcore_barrier()

        def body(nc, carry):
            pltpu.sync_copy(epk_hbm.at[r, t, nc, 1], dst_v)
            for b in range(CHK):
                pltpu.sync_copy(ones_v, acc.at[dst_v.at[b]], add=True)
            return carry
        lax.fori_loop(0, NCHK, body, None)
        plsc.subcore_barrier()
        _copy_out_chunk(t, acc, deg_hbm.at[r])
        plsc.subcore_barrier()


def _make_agg(r):
    @functools.partial(
        pl.kernel,
        out_type=jax.ShapeDtypeStruct((NSL, N, 16), jnp.float32),
        mesh=_MESH,
        compiler_params=pltpu.CompilerParams(use_tc_tiling_on_sc=False),
        scratch_types=[
            pltpu.VMEM((2, 2, CHK, BATCH), jnp.int32),
            pltpu.VMEM((2, CHK * BATCH, 16), jnp.float32),
            pltpu.VMEM((256, 16), jnp.float32),
            pltpu.VMEM_SHARED((ACC_ROWS, 16), jnp.float32),
            pltpu.SemaphoreType.DMA,
            pltpu.SemaphoreType.DMA,
            pltpu.SemaphoreType.DMA,
            pltpu.SemaphoreType.DMA,
        ],
        name=f"agg_etype{r}",
    )
    def _agg_r(x_hbm, epk_hbm, y_hbm,
               idx_v, rows_v, zeros_v, acc,
               gsem0, gsem1, ssem0, ssem1):
        c = lax.axis_index("c")
        t = lax.axis_index("s")
        _fill(zeros_v, 256, 0.0)

        def combo(q, carry):
            s = c * (NSL // NSC) + q
            xs = x_hbm.at[s]
            _zero_acc_chunk(zeros_v, acc, t * ACC_CHUNK)
            plsc.subcore_barrier()

            def fire_gathers(nc, p, gsem):
                pltpu.sync_copy(epk_hbm.at[r, t, nc], idx_v.at[p])
                for b in range(CHK):
                    pltpu.async_copy(
                        xs.at[idx_v.at[p, 0, b]],
                        rows_v.at[p, pl.ds(b * BATCH, BATCH)], gsem)

            def drain_scatter(nc, p, gsem, ssem):
                for b in range(CHK):
                    pltpu.make_async_copy(
                        xs.at[idx_v.at[p, 0, b]],
                        rows_v.at[p, pl.ds(b * BATCH, BATCH)], gsem).wait()
                for b in range(CHK):
                    pltpu.async_copy(rows_v.at[p, pl.ds(b * BATCH, BATCH)],
                                     acc.at[idx_v.at[p, 1, b]], ssem,
                                     add=True)
                for b in range(CHK):
                    pltpu.make_async_copy(
                        rows_v.at[p, pl.ds(b * BATCH, BATCH)],
                        acc.at[idx_v.at[p, 1, b]], ssem).wait()

            fire_gathers(0, 0, gsem0)

            def pair(i, carry2):
                base = 2 * i
                fire_gathers(base + 1, 1, gsem1)
                drain_scatter(base, 0, gsem0, ssem0)

                @pl.when(base + 2 < NCHK)
                def _():
                    fire_gathers(base + 2, 0, gsem0)
                drain_scatter(base + 1, 1, gsem1, ssem1)
                return carry2
            lax.fori_loop(0, NCHK // 2, pair, None)
            plsc.subcore_barrier()
            _copy_out_chunk(t, acc, y_hbm.at[s])
            plsc.subcore_barrier()
            return carry

        lax.fori_loop(0, NSL // NSC, combo, None)
    return _agg_r


_AGGS = [_make_agg(r) for r in range(R)]


@functools.partial(
    pl.kernel,
    out_type=jax.ShapeDtypeStruct((128, F), jnp.float32),
    mesh=_MESH,
    scratch_types=[
        pltpu.VMEM((BATCH,), jnp.int32),
        pltpu.VMEM((BATCH, F), jnp.float32),
        pltpu.SemaphoreType.DMA,
    ],
)
def _take_rows(x_hbm, idx_hbm, out_hbm, idx_v, rows_v, sem):
    c = lax.axis_index("c")
    t = lax.axis_index("s")

    @pl.when(jnp.logical_and(c == 0, t == 0))
    def _():
        pltpu.sync_copy(idx_hbm, idx_v)
        pltpu.async_copy(x_hbm.at[idx_v], rows_v, sem).wait()
        pltpu.sync_copy(rows_v, out_hbm)


def _layer_body(y0, y1, y2, y3, deg_ref, w_ref, b_ref, o_ref, *, act, sliced):
    acc = jnp.zeros((BN, F), jnp.float32)
    for r, y_ref in enumerate((y0, y1, y2, y3)):
        d = deg_ref[r, :, 0]
        rd = 1.0 / jnp.maximum(d, 1.0)
        yr = jnp.concatenate([y_ref[s] for s in range(NSL)], axis=1)
        yr = yr * rd[:, None]
        acc = acc + jnp.dot(yr, w_ref[r], preferred_element_type=jnp.float32)
        acc = acc + jnp.where(d > 0, 1.0, 0.0)[:, None] * b_ref[r][None, :]
    acc = act(acc)
    if sliced:
        for s in range(NSL):
            o_ref[s] = acc[:, s * 16:(s + 1) * 16]
    else:
        o_ref[...] = acc


def _layer_tc(Ys, deg, W, b, act, sliced):
    if sliced:
        out_spec = pl.BlockSpec((NSL, BN, 16), lambda i: (0, i, 0))
        out_shape = jax.ShapeDtypeStruct((NSL, N, 16), jnp.float32)
    else:
        out_spec = pl.BlockSpec((BN, F), lambda i: (i, 0))
        out_shape = jax.ShapeDtypeStruct((N, F), jnp.float32)
    y_spec = pl.BlockSpec((NSL, BN, 16), lambda i: (0, i, 0))
    return pl.pallas_call(
        functools.partial(_layer_body, act=act, sliced=sliced),
        grid=(NGRID,),
        in_specs=[
            y_spec, y_spec, y_spec, y_spec,
            pl.BlockSpec((R, BN, 16), lambda i: (0, i, 0)),
            pl.BlockSpec((R, F, F), lambda i: (0, 0, 0)),
            pl.BlockSpec((R, F), lambda i: (0, 0)),
        ],
        out_specs=out_spec,
        out_shape=out_shape,
    )(*Ys, deg, W, b)


def _slice_body(x_ref, o_ref):
    for s in range(NSL):
        o_ref[s] = x_ref[:, s * 16:(s + 1) * 16]


def _slice_tc(x):
    return pl.pallas_call(
        _slice_body,
        grid=(NGRID,),
        in_specs=[pl.BlockSpec((BN, F), lambda i: (i, 0))],
        out_specs=pl.BlockSpec((NSL, BN, 16), lambda i: (0, i, 0)),
        out_shape=jax.ShapeDtypeStruct((NSL, N, 16), jnp.float32),
    )(x)


def _lrelu(x):
    return jnp.where(x > 0, x, 0.01 * x)


def _final_body(ent_ref, rows_ref, wu_ref, bu_ref, o_ref):
    le = jax.nn.sigmoid(jnp.sum(rows_ref[:64], axis=0, keepdims=True))
    de = jax.nn.sigmoid(jnp.sum(rows_ref[64:], axis=0, keepdims=True))
    u = jax.nn.sigmoid(
        jnp.dot(le, wu_ref[:F], preferred_element_type=jnp.float32)
        + jnp.dot(de, wu_ref[F:], preferred_element_type=jnp.float32)
        + bu_ref[...])
    o_ref[...] = jnp.dot(ent_ref[...], u.reshape(F, 1),
                         preferred_element_type=jnp.float32)


def _final_tc(entity, rows, Wu, bu):
    return pl.pallas_call(
        _final_body,
        grid=(NGRID,),
        in_specs=[
            pl.BlockSpec((BN, F), lambda i: (i, 0)),
            pl.BlockSpec((128, F), lambda i: (0, 0)),
            pl.BlockSpec((2 * F, F), lambda i: (0, 0)),
            pl.BlockSpec((1, F), lambda i: (0, 0)),
        ],
        out_specs=pl.BlockSpec((BN, 1), lambda i: (i, 0)),
        out_shape=jax.ShapeDtypeStruct((N, 1), jnp.float32),
    )(entity, rows, Wu, bu)


def kernel(embed, W1, b1, W2, b2, W3, b3, Wu, bu,
           edge_src, edge_dst, liked_indices, disliked_indices):
    es = edge_src.astype(jnp.int32).reshape(R, NT, EP)
    ed = edge_dst.astype(jnp.int32).reshape(R, NT, EP)
    src_p = jnp.pad(es, ((0, 0), (0, 0), (0, EPP - EP))
                    ).reshape(R, NT, NCHK, CHK, BATCH)
    dst_p = jnp.pad(ed, ((0, 0), (0, 0), (0, EPP - EP)), constant_values=N
                    ).reshape(R, NT, NCHK, CHK, BATCH)
    epk = jnp.stack((src_p, dst_p), axis=3)

    deg = _deg(epk)

    x8 = _slice_tc(embed)

    def layer(x, W, b, act, sliced):
        Ys = tuple(_AGGS[r](x, epk) for r in range(R))
        return _layer_tc(Ys, deg, W, b, act, sliced)

    x8 = layer(x8, W1, b1, _lrelu, True)
    x8 = layer(x8, W2, b2, _lrelu, True)
    entity = layer(x8, W3, b3, jax.nn.sigmoid, False)

    idx = jnp.concatenate([liked_indices, disliked_indices]).astype(jnp.int32)
    rows = _take_rows(entity, idx)
    preds = _final_tc(entity, rows, Wu, bu.reshape(1, F))
    return preds.reshape(N)


# speedup vs baseline: 1.1029x; 1.0426x over previous
import functools

import jax
import jax.numpy as jnp
from jax import lax
from jax.experimental import pallas as pl
from jax.experimental.pallas import tpu as pltpu
from jax.experimental.pallas import tpu_sc as plsc

N = 100000
R = 4
E = 400000
F = 128
NSL = 8
NSC = 2
NT = 16
EP = E // NT
BATCH = 128
CHK = 5
NCHK = 40
NBAT = NCHK * CHK
EPP = NBAT * BATCH
ACC_CHUNK = 6264
ACC_ROWS = ACC_CHUNK * NT
OUT_CHUNK = 6248
OUT_LAST = N - 15 * OUT_CHUNK
BN = 1024
NGRID = (N + BN - 1) // BN

_MESH = plsc.VectorSubcoreMesh(
    core_axis_name="c", subcore_axis_name="s", num_cores=NSC, num_subcores=NT
)


def _fill(ref, nrows, value):
    def body(i, _):
        ref[i, :] = jnp.full((16,), value, jnp.float32)
        return _
    lax.fori_loop(0, nrows, body, None)


def _zero_acc_chunk(zeros_v, acc, base):
    def zbody(i, carry):
        pltpu.sync_copy(zeros_v, acc.at[pl.ds(base + i * 256, 256)])
        return carry
    lax.fori_loop(0, 24, zbody, None)
    pltpu.sync_copy(zeros_v.at[pl.ds(0, ACC_CHUNK - 6144)],
                    acc.at[pl.ds(base + 6144, ACC_CHUNK - 6144)])


def _copy_out_chunk(t, src_ref, dst_ref):
    obase = t * OUT_CHUNK

    @pl.when(t < NT - 1)
    def _():
        pltpu.sync_copy(src_ref.at[pl.ds(obase, OUT_CHUNK)],
                        dst_ref.at[pl.ds(obase, OUT_CHUNK)])

    @pl.when(t == NT - 1)
    def _():
        pltpu.sync_copy(src_ref.at[pl.ds((NT - 1) * OUT_CHUNK, OUT_LAST)],
                        dst_ref.at[pl.ds((NT - 1) * OUT_CHUNK, OUT_LAST)])


def _copy_out_chunk_packed(t, acc, dst_ref):
    obase = t * OUT_CHUNK

    @pl.when(t < NT - 1)
    def _():
        pltpu.sync_copy(
            acc.at[pl.ds(obase, OUT_CHUNK)].reshape(OUT_CHUNK // 8, F),
            dst_ref.at[pl.ds(obase // 8, OUT_CHUNK // 8)])

    @pl.when(t == NT - 1)
    def _():
        pltpu.sync_copy(
            acc.at[pl.ds((NT - 1) * OUT_CHUNK, OUT_LAST)].reshape(
                OUT_LAST // 8, F),
            dst_ref.at[pl.ds((NT - 1) * OUT_CHUNK // 8, OUT_LAST // 8)])


@functools.partial(
    pl.kernel,
    out_type=jax.ShapeDtypeStruct((R, N, 16), jnp.float32),
    mesh=_MESH,
    compiler_params=pltpu.CompilerParams(use_tc_tiling_on_sc=False),
    scratch_types=[
        pltpu.VMEM((CHK, BATCH), jnp.int32),
        pltpu.VMEM((BATCH, 16), jnp.float32),
        pltpu.VMEM((256, 16), jnp.float32),
        pltpu.VMEM_SHARED((ACC_ROWS, 16), jnp.float32),
    ],
)
def _deg(epk_hbm, deg_hbm, dst_v, ones_v, zeros_v, acc):
    c = lax.axis_index("c")
    t = lax.axis_index("s")
    _fill(zeros_v, 256, 0.0)
    _fill(ones_v, BATCH, 1.0)
    for rl in range(R // NSC):
        r = c * (R // NSC) + rl
        _zero_acc_chunk(zeros_v, acc, t * ACC_CHUNK)
        plsc.subcore_barrier()

        def body(nc, carry):
            pltpu.sync_copy(epk_hbm.at[r, t, nc, 1], dst_v)
            for b in range(CHK):
                pltpu.sync_copy(ones_v, acc.at[dst_v.at[b]], add=True)
            return carry
        lax.fori_loop(0, NCHK, body, None)
        plsc.subcore_barrier()
        _copy_out_chunk(t, acc, deg_hbm.at[r])
        plsc.subcore_barrier()


def _make_agg(r):
    @functools.partial(
        pl.kernel,
        out_type=jax.ShapeDtypeStruct((NSL, N, 16), jnp.float32),
        mesh=_MESH,
        compiler_params=pltpu.CompilerParams(use_tc_tiling_on_sc=False),
        scratch_types=[
            pltpu.VMEM((4, 2, CHK, BATCH), jnp.int32),
            pltpu.VMEM((2, CHK * BATCH, 16), jnp.float32),
            pltpu.VMEM((256, 16), jnp.float32),
            pltpu.VMEM_SHARED((ACC_ROWS, 16), jnp.float32),
            pltpu.SemaphoreType.DMA,
            pltpu.SemaphoreType.DMA,
            pltpu.SemaphoreType.DMA,
            pltpu.SemaphoreType.DMA,
            pltpu.SemaphoreType.DMA,
            pltpu.SemaphoreType.DMA,
        ],
        name=f"agg_etype{r}",
    )
    def _agg_r(x_hbm, epk_hbm, y_hbm,
               idx_v, rows_v, zeros_v, acc,
               gsem0, gsem1, ssem0, ssem1, isem0, isem1):
        c = lax.axis_index("c")
        t = lax.axis_index("s")
        _fill(zeros_v, 256, 0.0)

        def combo(q, carry):
            s = c * (NSL // NSC) + q
            xs = x_hbm.at[s]
            _zero_acc_chunk(zeros_v, acc, t * ACC_CHUNK)
            plsc.subcore_barrier()

            def stage_idx(nc, isem):
                pltpu.async_copy(epk_hbm.at[r, t, nc], idx_v.at[nc % 4], isem)

            def wait_idx(nc, isem):
                pltpu.make_async_copy(epk_hbm.at[r, t, nc],
                                      idx_v.at[nc % 4], isem).wait()

            def fire_gathers(nc, p):
                for b in range(CHK):
                    pltpu.async_copy(
                        xs.at[idx_v.at[nc % 4, 0, b]],
                        rows_v.at[p, pl.ds(b * BATCH, BATCH)],
                        gsem0 if p == 0 else gsem1)

            def drain_gathers(nc, p):
                for b in range(CHK):
                    pltpu.make_async_copy(
                        xs.at[idx_v.at[nc % 4, 0, b]],
                        rows_v.at[p, pl.ds(b * BATCH, BATCH)],
                        gsem0 if p == 0 else gsem1).wait()

            def fire_scatters(nc, p):
                for b in range(CHK):
                    pltpu.async_copy(rows_v.at[p, pl.ds(b * BATCH, BATCH)],
                                     acc.at[idx_v.at[nc % 4, 1, b]],
                                     ssem0 if p == 0 else ssem1, add=True)

            def drain_scatters(nc, p):
                for b in range(CHK):
                    pltpu.make_async_copy(
                        rows_v.at[p, pl.ds(b * BATCH, BATCH)],
                        acc.at[idx_v.at[nc % 4, 1, b]],
                        ssem0 if p == 0 else ssem1).wait()

            stage_idx(0, isem0)
            stage_idx(1, isem1)
            wait_idx(0, isem0)
            fire_gathers(0, 0)

            def pair(i, carry2):
                base = 2 * i
                wait_idx(base + 1, isem1)
                fire_gathers(base + 1, 1)

                @pl.when(base + 2 < NCHK)
                def _():
                    stage_idx(base + 2, isem0)
                drain_gathers(base, 0)
                fire_scatters(base, 0)

                @pl.when(base + 2 < NCHK)
                def _():
                    wait_idx(base + 2, isem0)
                drain_scatters(base, 0)

                @pl.when(base + 2 < NCHK)
                def _():
                    fire_gathers(base + 2, 0)

                @pl.when(base + 3 < NCHK)
                def _():
                    stage_idx(base + 3, isem1)
                drain_gathers(base + 1, 1)
                fire_scatters(base + 1, 1)
                drain_scatters(base + 1, 1)
                return carry2
            lax.fori_loop(0, NCHK // 2, pair, None)
            plsc.subcore_barrier()
            _copy_out_chunk(t, acc, y_hbm.at[s])
            plsc.subcore_barrier()
            return carry

        lax.fori_loop(0, NSL // NSC, combo, None)
    return _agg_r


_AGGS = [_make_agg(r) for r in range(R)]


@functools.partial(
    pl.kernel,
    out_type=jax.ShapeDtypeStruct((128, F), jnp.float32),
    mesh=_MESH,
    scratch_types=[
        pltpu.VMEM((BATCH,), jnp.int32),
        pltpu.VMEM((BATCH, F), jnp.float32),
        pltpu.SemaphoreType.DMA,
    ],
)
def _take_rows(x_hbm, idx_hbm, out_hbm, idx_v, rows_v, sem):
    c = lax.axis_index("c")
    t = lax.axis_index("s")

    @pl.when(jnp.logical_and(c == 0, t == 0))
    def _():
        pltpu.sync_copy(idx_hbm, idx_v)
        pltpu.async_copy(x_hbm.at[idx_v], rows_v, sem).wait()
        pltpu.sync_copy(rows_v, out_hbm)


def _layer_body(y0, y1, y2, y3, deg_ref, w_ref, b_ref, o_ref, *, act, sliced):
    acc = jnp.zeros((BN, F), jnp.float32)
    for r, y_ref in enumerate((y0, y1, y2, y3)):
        d = deg_ref[r, :, 0]
        rd = 1.0 / jnp.maximum(d, 1.0)
        yr = jnp.concatenate([y_ref[s] for s in range(NSL)], axis=1)
        yr = yr * rd[:, None]
        acc = acc + jnp.dot(yr, w_ref[r], preferred_element_type=jnp.float32)
        acc = acc + jnp.where(d > 0, 1.0, 0.0)[:, None] * b_ref[r][None, :]
    acc = act(acc)
    if sliced:
        for s in range(NSL):
            o_ref[s] = acc[:, s * 16:(s + 1) * 16]
    else:
        o_ref[...] = acc


def _layer_tc(Ys, deg, W, b, act, sliced):
    if sliced:
        out_spec = pl.BlockSpec((NSL, BN, 16), lambda i: (0, i, 0))
        out_shape = jax.ShapeDtypeStruct((NSL, N, 16), jnp.float32)
    else:
        out_spec = pl.BlockSpec((BN, F), lambda i: (i, 0))
        out_shape = jax.ShapeDtypeStruct((N, F), jnp.float32)
    y_spec = pl.BlockSpec((NSL, BN, 16), lambda i: (0, i, 0))
    return pl.pallas_call(
        functools.partial(_layer_body, act=act, sliced=sliced),
        grid=(NGRID,),
        in_specs=[
            y_spec, y_spec, y_spec, y_spec,
            pl.BlockSpec((R, BN, 16), lambda i: (0, i, 0)),
            pl.BlockSpec((R, F, F), lambda i: (0, 0, 0)),
            pl.BlockSpec((R, F), lambda i: (0, 0)),
        ],
        out_specs=out_spec,
        out_shape=out_shape,
    )(*Ys, deg, W, b)


def _slice_body(x_ref, o_ref):
    for s in range(NSL):
        o_ref[s] = x_ref[:, s * 16:(s + 1) * 16]


def _slice_tc(x):
    return pl.pallas_call(
        _slice_body,
        grid=(NGRID,),
        in_specs=[pl.BlockSpec((BN, F), lambda i: (i, 0))],
        out_specs=pl.BlockSpec((NSL, BN, 16), lambda i: (0, i, 0)),
        out_shape=jax.ShapeDtypeStruct((NSL, N, 16), jnp.float32),
    )(x)


def _lrelu(x):
    return jnp.where(x > 0, x, 0.01 * x)


def _final_body(ent_ref, rows_ref, wu_ref, bu_ref, o_ref):
    le = jax.nn.sigmoid(jnp.sum(rows_ref[:64], axis=0, keepdims=True))
    de = jax.nn.sigmoid(jnp.sum(rows_ref[64:], axis=0, keepdims=True))
    u = jax.nn.sigmoid(
        jnp.dot(le, wu_ref[:F], preferred_element_type=jnp.float32)
        + jnp.dot(de, wu_ref[F:], preferred_element_type=jnp.float32)
        + bu_ref[...])
    o_ref[...] = jnp.dot(ent_ref[...], u.reshape(F, 1),
                         preferred_element_type=jnp.float32)


def _final_tc(entity, rows, Wu, bu):
    return pl.pallas_call(
        _final_body,
        grid=(NGRID,),
        in_specs=[
            pl.BlockSpec((BN, F), lambda i: (i, 0)),
            pl.BlockSpec((128, F), lambda i: (0, 0)),
            pl.BlockSpec((2 * F, F), lambda i: (0, 0)),
            pl.BlockSpec((1, F), lambda i: (0, 0)),
        ],
        out_specs=pl.BlockSpec((BN, 1), lambda i: (i, 0)),
        out_shape=jax.ShapeDtypeStruct((N, 1), jnp.float32),
    )(entity, rows, Wu, bu)


def kernel(embed, W1, b1, W2, b2, W3, b3, Wu, bu,
           edge_src, edge_dst, liked_indices, disliked_indices):
    es = edge_src.astype(jnp.int32).reshape(R, NT, EP)
    ed = edge_dst.astype(jnp.int32).reshape(R, NT, EP)
    src_p = jnp.pad(es, ((0, 0), (0, 0), (0, EPP - EP))
                    ).reshape(R, NT, NCHK, CHK, BATCH)
    dst_p = jnp.pad(ed, ((0, 0), (0, 0), (0, EPP - EP)), constant_values=N
                    ).reshape(R, NT, NCHK, CHK, BATCH)
    epk = jnp.stack((src_p, dst_p), axis=3)

    deg = _deg(epk)

    x8 = _slice_tc(embed)

    def layer(x, W, b, act, sliced):
        Ys = tuple(_AGGS[r](x, epk) for r in range(R))
        return _layer_tc(Ys, deg, W, b, act, sliced)

    x8 = layer(x8, W1, b1, _lrelu, True)
    x8 = layer(x8, W2, b2, _lrelu, True)
    entity = layer(x8, W3, b3, jax.nn.sigmoid, False)

    idx = jnp.concatenate([liked_indices, disliked_indices]).astype(jnp.int32)
    rows = _take_rows(entity, idx)
    preds = _final_tc(entity, rows, Wu, bu.reshape(1, F))
    return preds.reshape(N)


# speedup vs baseline: 1.1079x; 1.0045x over previous
import functools

import jax
import jax.numpy as jnp
from jax import lax
from jax.experimental import pallas as pl
from jax.experimental.pallas import tpu as pltpu
from jax.experimental.pallas import tpu_sc as plsc

N = 100000
R = 4
E = 400000
F = 128
NSL = 8
NSC = 2
NT = 16
EP = E // NT
BATCH = 128
CHK = 5
NCHK = 40
NBAT = NCHK * CHK
EPP = NBAT * BATCH
ACC_CHUNK = 6264
ACC_ROWS = ACC_CHUNK * NT
OUT_CHUNK = 6248
OUT_LAST = N - 15 * OUT_CHUNK
BN = 1024
NGRID = (N + BN - 1) // BN

_MESH = plsc.VectorSubcoreMesh(
    core_axis_name="c", subcore_axis_name="s", num_cores=NSC, num_subcores=NT
)


def _fill(ref, nrows, value):
    def body(i, _):
        ref[i, :] = jnp.full((16,), value, jnp.float32)
        return _
    lax.fori_loop(0, nrows, body, None)


def _zero_acc_chunk(zeros_v, acc, base, sem=None):
    if sem is None:
        def zbody(i, carry):
            pltpu.sync_copy(zeros_v, acc.at[pl.ds(base + i * 256, 256)])
            return carry
        lax.fori_loop(0, 24, zbody, None)
        pltpu.sync_copy(zeros_v.at[pl.ds(0, ACC_CHUNK - 6144)],
                        acc.at[pl.ds(base + 6144, ACC_CHUNK - 6144)])
        return

    def zfire(i, carry):
        pltpu.async_copy(zeros_v, acc.at[pl.ds(base + i * 256, 256)], sem)
        return carry
    lax.fori_loop(0, 24, zfire, None)
    pltpu.async_copy(zeros_v.at[pl.ds(0, ACC_CHUNK - 6144)],
                     acc.at[pl.ds(base + 6144, ACC_CHUNK - 6144)], sem)

    def zdrain(i, carry):
        pltpu.make_async_copy(
            zeros_v, acc.at[pl.ds(base + i * 256, 256)], sem).wait()
        return carry
    lax.fori_loop(0, 24, zdrain, None)
    pltpu.make_async_copy(
        zeros_v.at[pl.ds(0, ACC_CHUNK - 6144)],
        acc.at[pl.ds(base + 6144, ACC_CHUNK - 6144)], sem).wait()


def _copy_out_chunk(t, src_ref, dst_ref):
    obase = t * OUT_CHUNK

    @pl.when(t < NT - 1)
    def _():
        pltpu.sync_copy(src_ref.at[pl.ds(obase, OUT_CHUNK)],
                        dst_ref.at[pl.ds(obase, OUT_CHUNK)])

    @pl.when(t == NT - 1)
    def _():
        pltpu.sync_copy(src_ref.at[pl.ds((NT - 1) * OUT_CHUNK, OUT_LAST)],
                        dst_ref.at[pl.ds((NT - 1) * OUT_CHUNK, OUT_LAST)])


def _copy_out_chunk_packed(t, acc, dst_ref):
    obase = t * OUT_CHUNK

    @pl.when(t < NT - 1)
    def _():
        pltpu.sync_copy(
            acc.at[pl.ds(obase, OUT_CHUNK)].reshape(OUT_CHUNK // 8, F),
            dst_ref.at[pl.ds(obase // 8, OUT_CHUNK // 8)])

    @pl.when(t == NT - 1)
    def _():
        pltpu.sync_copy(
            acc.at[pl.ds((NT - 1) * OUT_CHUNK, OUT_LAST)].reshape(
                OUT_LAST // 8, F),
            dst_ref.at[pl.ds((NT - 1) * OUT_CHUNK // 8, OUT_LAST // 8)])


@functools.partial(
    pl.kernel,
    out_type=jax.ShapeDtypeStruct((R, N, 16), jnp.float32),
    mesh=_MESH,
    compiler_params=pltpu.CompilerParams(use_tc_tiling_on_sc=False),
    scratch_types=[
        pltpu.VMEM((CHK, BATCH), jnp.int32),
        pltpu.VMEM((BATCH, 16), jnp.float32),
        pltpu.VMEM((256, 16), jnp.float32),
        pltpu.VMEM_SHARED((ACC_ROWS, 16), jnp.float32),
    ],
)
def _deg(epk_hbm, deg_hbm, dst_v, ones_v, zeros_v, acc):
    c = lax.axis_index("c")
    t = lax.axis_index("s")
    _fill(zeros_v, 256, 0.0)
    _fill(ones_v, BATCH, 1.0)
    for rl in range(R // NSC):
        r = c * (R // NSC) + rl
        _zero_acc_chunk(zeros_v, acc, t * ACC_CHUNK)
        plsc.subcore_barrier()

        def body(nc, carry):
            pltpu.sync_copy(epk_hbm.at[r, t, nc, 1], dst_v)
            for b in range(CHK):
                pltpu.sync_copy(ones_v, acc.at[dst_v.at[b]], add=True)
            return carry
        lax.fori_loop(0, NCHK, body, None)
        plsc.subcore_barrier()
        _copy_out_chunk(t, acc, deg_hbm.at[r])
        plsc.subcore_barrier()


def _make_agg(r):
    @functools.partial(
        pl.kernel,
        out_type=jax.ShapeDtypeStruct((NSL, N, 16), jnp.float32),
        mesh=_MESH,
        compiler_params=pltpu.CompilerParams(use_tc_tiling_on_sc=False),
        scratch_types=[
            pltpu.VMEM((4, 2, CHK, BATCH), jnp.int32),
            pltpu.VMEM((2, CHK * BATCH, 16), jnp.float32),
            pltpu.VMEM((256, 16), jnp.float32),
            pltpu.VMEM_SHARED((ACC_ROWS, 16), jnp.float32),
            pltpu.SemaphoreType.DMA,
            pltpu.SemaphoreType.DMA,
            pltpu.SemaphoreType.DMA,
            pltpu.SemaphoreType.DMA,
            pltpu.SemaphoreType.DMA,
            pltpu.SemaphoreType.DMA,
        ],
        name=f"agg_etype{r}",
    )
    def _agg_r(x_hbm, epk_hbm, y_hbm,
               idx_v, rows_v, zeros_v, acc,
               gsem0, gsem1, ssem0, ssem1, isem0, isem1):
        c = lax.axis_index("c")
        t = lax.axis_index("s")
        _fill(zeros_v, 256, 0.0)

        def combo(q, carry):
            s = c * (NSL // NSC) + q
            xs = x_hbm.at[s]
            _zero_acc_chunk(zeros_v, acc, t * ACC_CHUNK, gsem0)
            plsc.subcore_barrier()

            def stage_idx(nc, isem):
                pltpu.async_copy(epk_hbm.at[r, t, nc], idx_v.at[nc % 4], isem)

            def wait_idx(nc, isem):
                pltpu.make_async_copy(epk_hbm.at[r, t, nc],
                                      idx_v.at[nc % 4], isem).wait()

            def fire_gathers(nc, p):
                for b in range(CHK):
                    pltpu.async_copy(
                        xs.at[idx_v.at[nc % 4, 0, b]],
                        rows_v.at[p, pl.ds(b * BATCH, BATCH)],
                        gsem0 if p == 0 else gsem1)

            def drain_gathers(nc, p):
                for b in range(CHK):
                    pltpu.make_async_copy(
                        xs.at[idx_v.at[nc % 4, 0, b]],
                        rows_v.at[p, pl.ds(b * BATCH, BATCH)],
                        gsem0 if p == 0 else gsem1).wait()

            def fire_scatters(nc, p):
                for b in range(CHK):
                    pltpu.async_copy(rows_v.at[p, pl.ds(b * BATCH, BATCH)],
                                     acc.at[idx_v.at[nc % 4, 1, b]],
                                     ssem0 if p == 0 else ssem1, add=True)

            def drain_scatters(nc, p):
                for b in range(CHK):
                    pltpu.make_async_copy(
                        rows_v.at[p, pl.ds(b * BATCH, BATCH)],
                        acc.at[idx_v.at[nc % 4, 1, b]],
                        ssem0 if p == 0 else ssem1).wait()

            stage_idx(0, isem0)
            stage_idx(1, isem1)
            wait_idx(0, isem0)
            fire_gathers(0, 0)

            def pair(i, carry2):
                base = 2 * i
                wait_idx(base + 1, isem1)
                fire_gathers(base + 1, 1)

                @pl.when(base + 2 < NCHK)
                def _():
                    stage_idx(base + 2, isem0)
                drain_gathers(base, 0)
                fire_scatters(base, 0)

                @pl.when(base + 2 < NCHK)
                def _():
                    wait_idx(base + 2, isem0)
                drain_scatters(base, 0)

                @pl.when(base + 2 < NCHK)
                def _():
                    fire_gathers(base + 2, 0)

                @pl.when(base + 3 < NCHK)
                def _():
                    stage_idx(base + 3, isem1)
                drain_gathers(base + 1, 1)
                fire_scatters(base + 1, 1)
                drain_scatters(base + 1, 1)
                return carry2
            lax.fori_loop(0, NCHK // 2, pair, None)
            plsc.subcore_barrier()
            _copy_out_chunk(t, acc, y_hbm.at[s])
            plsc.subcore_barrier()
            return carry

        lax.fori_loop(0, NSL // NSC, combo, None)
    return _agg_r


_AGGS = [_make_agg(r) for r in range(R)]


@functools.partial(
    pl.kernel,
    out_type=jax.ShapeDtypeStruct((128, F), jnp.float32),
    mesh=_MESH,
    scratch_types=[
        pltpu.VMEM((BATCH,), jnp.int32),
        pltpu.VMEM((BATCH, F), jnp.float32),
        pltpu.SemaphoreType.DMA,
    ],
)
def _take_rows(x_hbm, idx_hbm, out_hbm, idx_v, rows_v, sem):
    c = lax.axis_index("c")
    t = lax.axis_index("s")

    @pl.when(jnp.logical_and(c == 0, t == 0))
    def _():
        pltpu.sync_copy(idx_hbm, idx_v)
        pltpu.async_copy(x_hbm.at[idx_v], rows_v, sem).wait()
        pltpu.sync_copy(rows_v, out_hbm)


def _layer_body(y0, y1, y2, y3, deg_ref, w_ref, b_ref, o_ref, *, act, sliced):
    acc = jnp.zeros((BN, F), jnp.float32)
    for r, y_ref in enumerate((y0, y1, y2, y3)):
        d = deg_ref[r, :, 0]
        rd = 1.0 / jnp.maximum(d, 1.0)
        yr = jnp.concatenate([y_ref[s] for s in range(NSL)], axis=1)
        yr = yr * rd[:, None]
        acc = acc + jnp.dot(yr, w_ref[r], preferred_element_type=jnp.float32)
        acc = acc + jnp.where(d > 0, 1.0, 0.0)[:, None] * b_ref[r][None, :]
    acc = act(acc)
    if sliced:
        for s in range(NSL):
            o_ref[s] = acc[:, s * 16:(s + 1) * 16]
    else:
        o_ref[...] = acc


def _layer_tc(Ys, deg, W, b, act, sliced):
    if sliced:
        out_spec = pl.BlockSpec((NSL, BN, 16), lambda i: (0, i, 0))
        out_shape = jax.ShapeDtypeStruct((NSL, N, 16), jnp.float32)
    else:
        out_spec = pl.BlockSpec((BN, F), lambda i: (i, 0))
        out_shape = jax.ShapeDtypeStruct((N, F), jnp.float32)
    y_spec = pl.BlockSpec((NSL, BN, 16), lambda i: (0, i, 0))
    return pl.pallas_call(
        functools.partial(_layer_body, act=act, sliced=sliced),
        grid=(NGRID,),
        in_specs=[
            y_spec, y_spec, y_spec, y_spec,
            pl.BlockSpec((R, BN, 16), lambda i: (0, i, 0)),
            pl.BlockSpec((R, F, F), lambda i: (0, 0, 0)),
            pl.BlockSpec((R, F), lambda i: (0, 0)),
        ],
        out_specs=out_spec,
        out_shape=out_shape,
    )(*Ys, deg, W, b)


def _slice_body(x_ref, o_ref):
    for s in range(NSL):
        o_ref[s] = x_ref[:, s * 16:(s + 1) * 16]


def _slice_tc(x):
    return pl.pallas_call(
        _slice_body,
        grid=(NGRID,),
        in_specs=[pl.BlockSpec((BN, F), lambda i: (i, 0))],
        out_specs=pl.BlockSpec((NSL, BN, 16), lambda i: (0, i, 0)),
        out_shape=jax.ShapeDtypeStruct((NSL, N, 16), jnp.float32),
    )(x)


def _lrelu(x):
    return jnp.where(x > 0, x, 0.01 * x)


def _final_body(ent_ref, rows_ref, wu_ref, bu_ref, o_ref):
    le = jax.nn.sigmoid(jnp.sum(rows_ref[:64], axis=0, keepdims=True))
    de = jax.nn.sigmoid(jnp.sum(rows_ref[64:], axis=0, keepdims=True))
    u = jax.nn.sigmoid(
        jnp.dot(le, wu_ref[:F], preferred_element_type=jnp.float32)
        + jnp.dot(de, wu_ref[F:], preferred_element_type=jnp.float32)
        + bu_ref[...])
    o_ref[...] = jnp.dot(ent_ref[...], u.reshape(F, 1),
                         preferred_element_type=jnp.float32)


def _final_tc(entity, rows, Wu, bu):
    return pl.pallas_call(
        _final_body,
        grid=(NGRID,),
        in_specs=[
            pl.BlockSpec((BN, F), lambda i: (i, 0)),
            pl.BlockSpec((128, F), lambda i: (0, 0)),
            pl.BlockSpec((2 * F, F), lambda i: (0, 0)),
            pl.BlockSpec((1, F), lambda i: (0, 0)),
        ],
        out_specs=pl.BlockSpec((BN, 1), lambda i: (i, 0)),
        out_shape=jax.ShapeDtypeStruct((N, 1), jnp.float32),
    )(entity, rows, Wu, bu)


def kernel(embed, W1, b1, W2, b2, W3, b3, Wu, bu,
           edge_src, edge_dst, liked_indices, disliked_indices):
    es = edge_src.astype(jnp.int32).reshape(R, NT, EP)
    ed = edge_dst.astype(jnp.int32).reshape(R, NT, EP)
    src_p = jnp.pad(es, ((0, 0), (0, 0), (0, EPP - EP))
                    ).reshape(R, NT, NCHK, CHK, BATCH)
    dst_p = jnp.pad(ed, ((0, 0), (0, 0), (0, EPP - EP)), constant_values=N
                    ).reshape(R, NT, NCHK, CHK, BATCH)
    epk = jnp.stack((src_p, dst_p), axis=3)

    deg = _deg(epk)

    x8 = _slice_tc(embed)

    def layer(x, W, b, act, sliced):
        Ys = tuple(_AGGS[r](x, epk) for r in range(R))
        return _layer_tc(Ys, deg, W, b, act, sliced)

    x8 = layer(x8, W1, b1, _lrelu, True)
    x8 = layer(x8, W2, b2, _lrelu, True)
    entity = layer(x8, W3, b3, jax.nn.sigmoid, False)

    idx = jnp.concatenate([liked_indices, disliked_indices]).astype(jnp.int32)
    rows = _take_rows(entity, idx)
    preds = _final_tc(entity, rows, Wu, bu.reshape(1, F))
    return preds.reshape(N)
